# baseline (speedup 1.0000x reference)
import functools

import jax
import jax.numpy as jnp
from jax import lax
from jax.experimental import pallas as pl
from jax.experimental.pallas import tpu as pltpu
from jax.experimental.pallas import tpu_sc as plsc

NC = 2
NS = 16
NW = NC * NS
DEGW = 16


def _sc_mesh():
    return plsc.VectorSubcoreMesh(core_axis_name="c", subcore_axis_name="s",
                                  num_cores=NC, num_subcores=NS)


def _sc_degree(npad, e, k, grp):
    epw = e // NW
    ch = epw // k
    rpt = npad // NS

    ng = ch // grp
    assert ng * grp == ch and ng >= 2

    scratch = (
        [pltpu.VMEM((k,), jnp.int32) for _ in range(2 * grp)]
        + [pltpu.VMEM((k, DEGW), jnp.float32),
           pltpu.VMEM_SHARED((npad, DEGW), jnp.float32)]
        + [pltpu.SemaphoreType.DMA] * 2
    )

    @functools.partial(
        pl.kernel, mesh=_sc_mesh(),
        out_type=jax.ShapeDtypeStruct((NC, npad, DEGW), jnp.float32),
        compiler_params=pltpu.CompilerParams(use_tc_tiling_on_sc=False),
        scratch_types=scratch,
    )
    def deg_kernel(dst_hbm, ones_hbm, zeros_hbm, out_hbm, *refs):
        dstv = refs[0:2 * grp]
        ones_v = refs[2 * grp]
        acc_sh = refs[2 * grp + 1]
        isem = refs[2 * grp + 2:2 * grp + 4]
        cid = lax.axis_index("c")
        sid = lax.axis_index("s")
        wid = cid * NS + sid
        rbase = sid * rpt
        ebase = wid * epw

        def fire_idx(n, s):
            for j in range(grp):
                off = ebase + (n * grp + j) * k
                pltpu.async_copy(dst_hbm.at[pl.ds(off, k)], dstv[s * grp + j], isem[s])

        def drain_idx(s):
            for j in range(grp):
                pltpu.make_async_copy(dst_hbm.at[pl.ds(0, k)], dstv[s * grp + j], isem[s]).wait()

        def scatters(s):
            for j in range(grp):
                pltpu.sync_copy(ones_v, acc_sh.at[dstv[s * grp + j]], add=True)

        pltpu.sync_copy(ones_hbm, ones_v)
        pltpu.sync_copy(zeros_hbm.at[pl.ds(rbase, rpt)], acc_sh.at[pl.ds(rbase, rpt)])
        plsc.subcore_barrier()

        def group(n, p, has_next):
            if has_next:
                fire_idx(n + 1, 1 - p)
            drain_idx(p)
            scatters(p)

        fire_idx(0, 0)

        def body(t, carry):
            group(2 * t, 0, True)
            group(2 * t + 1, 1, True)
            return carry

        nt = 1 if ng % 2 == 1 else 2
        lax.fori_loop(0, (ng - nt) // 2, body, 0)
        for i in range(nt):
            n = ng - nt + i
            group(n, n % 2, i < nt - 1)

        plsc.subcore_barrier()
        pltpu.sync_copy(acc_sh.at[pl.ds(rbase, rpt)],
                        out_hbm.at[cid, pl.ds(rbase, rpt)])

    return deg_kernel


def _sc_scatter(npad, e, d, k, grp):
    epw = e // NW
    ch = epw // k
    ng = ch // grp
    assert ng * grp == ch and ng >= 4
    rpt = npad // NS

    scratch = (
        [pltpu.VMEM((k,), jnp.int32) for _ in range(2 * grp)]
        + [pltpu.VMEM((k,), jnp.int32) for _ in range(2 * grp)]
        + [pltpu.VMEM((k, d), jnp.float32) for _ in range(2 * grp)]
        + [pltpu.VMEM_SHARED((npad, d), jnp.float32)]
        + [pltpu.SemaphoreType.DMA] * 4
    )

    @functools.partial(
        pl.kernel, mesh=_sc_mesh(),
        out_type=jax.ShapeDtypeStruct((NC, npad, d), jnp.float32),
        compiler_params=pltpu.CompilerParams(use_tc_tiling_on_sc=(d % 128 == 0)),
        scratch_types=scratch,
    )
    def scat_kernel(src_hbm, dst_hbm, rows_hbm, zeros_hbm, out_hbm, *refs):
        srcv = refs[0:2 * grp]
        dstv = refs[2 * grp:4 * grp]
        bufs = refs[4 * grp:6 * grp]
        acc_sh = refs[6 * grp]
        isem = refs[6 * grp + 1:6 * grp + 3]
        gsem = refs[6 * grp + 3:6 * grp + 5]
        cid = lax.axis_index("c")
        sid = lax.axis_index("s")
        wid = cid * NS + sid
        rbase = sid * rpt
        ebase = wid * epw

        def fire_idx(n, s):
            for j in range(grp):
                off = ebase + (n * grp + j) * k
                pltpu.async_copy(src_hbm.at[pl.ds(off, k)], srcv[s * grp + j], isem[s])
                pltpu.async_copy(dst_hbm.at[pl.ds(off, k)], dstv[s * grp + j], isem[s])

        def drain_idx(s):
            for j in range(grp):
                pltpu.make_async_copy(src_hbm.at[pl.ds(0, k)], srcv[s * grp + j], isem[s]).wait()
                pltpu.make_async_copy(dst_hbm.at[pl.ds(0, k)], dstv[s * grp + j], isem[s]).wait()

        def fire_gather(s):
            for j in range(grp):
                pltpu.async_copy(rows_hbm.at[srcv[s * grp + j]], bufs[s * grp + j], gsem[s])

        def drain_gather(s):
            for j in range(grp):
                pltpu.make_async_copy(rows_hbm.at[srcv[s * grp + j]], bufs[s * grp + j], gsem[s]).wait()

        def scatters(s):
            for j in range(grp):
                pltpu.sync_copy(bufs[s * grp + j], acc_sh.at[dstv[s * grp + j]], add=True)

        pltpu.sync_copy(zeros_hbm.at[pl.ds(rbase, rpt)], acc_sh.at[pl.ds(rbase, rpt)])
        plsc.subcore_barrier()

        fire_idx(0, 0)
        fire_idx(1, 1)
        drain_idx(0)
        fire_gather(0)

        def group(n, p, has_next, has_next2):
            if has_next:
                drain_idx(1 - p)
                fire_gather(1 - p)
            drain_gather(p)
            scatters(p)
            if has_next2:
                fire_idx(n + 2, p)

        def body(t, carry):
            group(2 * t, 0, True, True)
            group(2 * t + 1, 1, True, True)
            return carry

        nt = 3 if ng % 2 == 1 else 4
        lax.fori_loop(0, (ng - nt) // 2, body, 0)
        n0 = ng - nt
        for i in range(nt):
            n = n0 + i
            group(n, n % 2, i < nt - 1, i < nt - 2)

        plsc.subcore_barrier()
        pltpu.sync_copy(acc_sh.at[pl.ds(rbase, rpt)],
                        out_hbm.at[cid, pl.ds(rbase, rpt)])

    return scat_kernel


def _tc1_body(x_ref, w_ref, d0_ref, d1_ref, xws_ref, dinv_ref):
    deg = d0_ref[...] + d1_ref[...] + 1.0
    dv = lax.rsqrt(deg)
    xw = jnp.dot(x_ref[...], w_ref[...], preferred_element_type=jnp.float32)
    xws_ref[...] = xw * dv[:, 0:1]
    dinv_ref[...] = dv


def _tc1(n, d_in, d_hid, bt):
    return pl.pallas_call(
        _tc1_body,
        grid=(n // bt,),
        in_specs=[
            pl.BlockSpec((bt, d_in), lambda i: (i, 0)),
            pl.BlockSpec((d_in, d_hid), lambda i: (0, 0)),
            pl.BlockSpec((bt, DEGW), lambda i: (i, 0)),
            pl.BlockSpec((bt, DEGW), lambda i: (i, 0)),
        ],
        out_specs=[
            pl.BlockSpec((bt, d_hid), lambda i: (i, 0)),
            pl.BlockSpec((bt, DEGW), lambda i: (i, 0)),
        ],
        out_shape=[
            jax.ShapeDtypeStruct((n, d_hid), jnp.float32),
            jax.ShapeDtypeStruct((n, DEGW), jnp.float32),
        ],
    )


def _tc2_body(p0_ref, p1_ref, xws_ref, dinv_ref, b1_ref, wcat_ref, out_ref):
    dv = dinv_ref[...][:, 0:1]
    s = (p0_ref[...] + p1_ref[...] + xws_ref[...]) * dv + b1_ref[...]
    h = jnp.maximum(s, 0.0)
    out_ref[...] = jnp.dot(h, wcat_ref[...], preferred_element_type=jnp.float32) * dv


def _tc2(n, d_hid, d_cat, bt):
    return pl.pallas_call(
        _tc2_body,
        grid=(n // bt,),
        in_specs=[
            pl.BlockSpec((bt, d_hid), lambda i: (i, 0)),
            pl.BlockSpec((bt, d_hid), lambda i: (i, 0)),
            pl.BlockSpec((bt, d_hid), lambda i: (i, 0)),
            pl.BlockSpec((bt, DEGW), lambda i: (i, 0)),
            pl.BlockSpec((1, d_hid), lambda i: (0, 0)),
            pl.BlockSpec((d_hid, d_cat), lambda i: (0, 0)),
        ],
        out_specs=pl.BlockSpec((bt, d_cat), lambda i: (i, 0)),
        out_shape=jax.ShapeDtypeStruct((n, d_cat), jnp.float32),
    )


def _tc3_body(q0_ref, q1_ref, hcs_ref, dinv_ref, bm_ref, bl_ref, mu_ref, ls_ref):
    d_lat = mu_ref.shape[1]
    dv = dinv_ref[...][:, 0:1]
    o = (q0_ref[...] + q1_ref[...] + hcs_ref[...]) * dv
    mu_ref[...] = o[:, :d_lat] + bm_ref[...]
    ls_ref[...] = o[:, d_lat:] + bl_ref[...]


def _tc3(n, d_cat, d_lat, bt):
    return pl.pallas_call(
        _tc3_body,
        grid=(n // bt,),
        in_specs=[
            pl.BlockSpec((bt, d_cat), lambda i: (i, 0)),
            pl.BlockSpec((bt, d_cat), lambda i: (i, 0)),
            pl.BlockSpec((bt, d_cat), lambda i: (i, 0)),
            pl.BlockSpec((bt, DEGW), lambda i: (i, 0)),
            pl.BlockSpec((1, d_lat), lambda i: (0, 0)),
            pl.BlockSpec((1, d_lat), lambda i: (0, 0)),
        ],
        out_specs=[
            pl.BlockSpec((bt, d_lat), lambda i: (i, 0)),
            pl.BlockSpec((bt, d_lat), lambda i: (i, 0)),
        ],
        out_shape=[
            jax.ShapeDtypeStruct((n, d_lat), jnp.float32),
            jax.ShapeDtypeStruct((n, d_lat), jnp.float32),
        ],
    )


def kernel(x, edge_index, W1, b1, W_mu, b_mu, W_logstd, b_logstd):
    n, d_in = x.shape
    d_hid = W1.shape[1]
    d_lat = W_mu.shape[1]
    d_cat = 2 * d_lat
    e = edge_index.shape[1]
    bt = 1000
    gran = 8 * NS
    npad = ((n + gran - 1) // gran) * gran

    kc = 128
    epw_pad = ((e // NW + 2 * kc - 1) // (2 * kc)) * (2 * kc)
    e_pad = epw_pad * NW
    pad = e_pad - e
    src = jnp.concatenate([edge_index[0].astype(jnp.int32),
                           jnp.zeros((pad,), jnp.int32)])
    dst = jnp.concatenate([edge_index[1].astype(jnp.int32),
                           n + (jnp.arange(pad, dtype=jnp.int32) % (npad - n))])

    ones_k = jnp.ones((kc, DEGW), jnp.float32)
    zeros_deg = jnp.zeros((npad, DEGW), jnp.float32)
    degp = _sc_degree(npad, e_pad, kc, 2)(dst, ones_k, zeros_deg)

    xws, dinv = _tc1(n, d_in, d_hid, bt)(x, W1, degp[0, :n], degp[1, :n])

    zeros_h = jnp.zeros((npad, d_hid), jnp.float32)
    p = _sc_scatter(npad, e_pad, d_hid, kc, 1)(src, dst, xws, zeros_h)

    wcat = jnp.concatenate([W_mu, W_logstd], axis=1)
    hcs = _tc2(n, d_hid, d_cat, bt)(p[0, :n], p[1, :n], xws, dinv,
                                    b1.reshape(1, d_hid), wcat)

    zeros_c = jnp.zeros((npad, d_cat), jnp.float32)
    q = _sc_scatter(npad, e_pad, d_cat, kc, 2)(src, dst, hcs, zeros_c)

    mu, logstd = _tc3(n, d_cat, d_lat, bt)(q[0, :n], q[1, :n], hcs, dinv,
                                           b_mu.reshape(1, d_lat),
                                           b_logstd.reshape(1, d_lat))
    return (mu, logstd)


# speedup vs baseline: 1.5844x; 1.5844x over previous
import functools

import jax
import jax.numpy as jnp
from jax import lax
from jax.experimental import pallas as pl
from jax.experimental.pallas import tpu as pltpu
from jax.experimental.pallas import tpu_sc as plsc

NC = 2
NS = 16
NW = NC * NS
DEGW = 16


def _sc_mesh():
    return plsc.VectorSubcoreMesh(core_axis_name="c", subcore_axis_name="s",
                                  num_cores=NC, num_subcores=NS)


def _sc_degree(npad, e, k, grp):
    epw = e // NW
    ch = epw // k
    rpt = npad // NS

    ng = ch // grp
    assert ng * grp == ch and ng >= 2

    scratch = (
        [pltpu.VMEM((k,), jnp.int32) for _ in range(2 * grp)]
        + [pltpu.VMEM((k, DEGW), jnp.float32),
           pltpu.VMEM_SHARED((npad, DEGW), jnp.float32)]
        + [pltpu.SemaphoreType.DMA] * 2
    )

    @functools.partial(
        pl.kernel, mesh=_sc_mesh(),
        out_type=jax.ShapeDtypeStruct((NC, npad, DEGW), jnp.float32),
        compiler_params=pltpu.CompilerParams(use_tc_tiling_on_sc=False),
        scratch_types=scratch,
    )
    def deg_kernel(dst_hbm, ones_hbm, zeros_hbm, out_hbm, *refs):
        dstv = refs[0:2 * grp]
        ones_v = refs[2 * grp]
        acc_sh = refs[2 * grp + 1]
        isem = refs[2 * grp + 2:2 * grp + 4]
        cid = lax.axis_index("c")
        sid = lax.axis_index("s")
        wid = cid * NS + sid
        rbase = sid * rpt
        ebase = wid * epw

        def fire_idx(n, s):
            for j in range(grp):
                off = ebase + (n * grp + j) * k
                pltpu.async_copy(dst_hbm.at[pl.ds(off, k)], dstv[s * grp + j], isem[s])

        def drain_idx(s):
            for j in range(grp):
                pltpu.make_async_copy(dst_hbm.at[pl.ds(0, k)], dstv[s * grp + j], isem[s]).wait()

        def scatters(s):
            for j in range(grp):
                pltpu.sync_copy(ones_v, acc_sh.at[dstv[s * grp + j]], add=True)

        pltpu.sync_copy(ones_hbm, ones_v)
        pltpu.sync_copy(zeros_hbm.at[pl.ds(rbase, rpt)], acc_sh.at[pl.ds(rbase, rpt)])
        plsc.subcore_barrier()

        def group(n, p, has_next):
            if has_next:
                fire_idx(n + 1, 1 - p)
            drain_idx(p)
            scatters(p)

        fire_idx(0, 0)

        def body(t, carry):
            group(2 * t, 0, True)
            group(2 * t + 1, 1, True)
            return carry

        nt = 1 if ng % 2 == 1 else 2
        lax.fori_loop(0, (ng - nt) // 2, body, 0)
        for i in range(nt):
            n = ng - nt + i
            group(n, n % 2, i < nt - 1)

        plsc.subcore_barrier()
        pltpu.sync_copy(acc_sh.at[pl.ds(rbase, rpt)],
                        out_hbm.at[cid, pl.ds(rbase, rpt)])

    return deg_kernel


def _sc_scatter(npad, e, d, k, grp):
    epw = e // NW
    ch = epw // k
    ng = ch // grp
    assert ng * grp == ch and ng >= 4
    rpt = npad // NS

    scratch = (
        [pltpu.VMEM((k,), jnp.int32) for _ in range(2 * grp)]
        + [pltpu.VMEM((k,), jnp.int32) for _ in range(2 * grp)]
        + [pltpu.VMEM((k, d), jnp.float32) for _ in range(2 * grp)]
        + [pltpu.VMEM_SHARED((npad, d), jnp.float32)]
        + [pltpu.SemaphoreType.DMA] * 4
    )

    @functools.partial(
        pl.kernel, mesh=_sc_mesh(),
        out_type=jax.ShapeDtypeStruct((NC, npad, d), jnp.float32),
        compiler_params=pltpu.CompilerParams(use_tc_tiling_on_sc=(d % 128 == 0)),
        scratch_types=scratch,
    )
    def scat_kernel(src_hbm, dst_hbm, rows_hbm, zeros_hbm, out_hbm, *refs):
        srcv = refs[0:2 * grp]
        dstv = refs[2 * grp:4 * grp]
        bufs = refs[4 * grp:6 * grp]
        acc_sh = refs[6 * grp]
        isem = refs[6 * grp + 1:6 * grp + 3]
        gsem = refs[6 * grp + 3:6 * grp + 5]
        cid = lax.axis_index("c")
        sid = lax.axis_index("s")
        wid = cid * NS + sid
        rbase = sid * rpt
        ebase = wid * epw

        def fire_idx(n, s):
            for j in range(grp):
                off = ebase + (n * grp + j) * k
                pltpu.async_copy(src_hbm.at[pl.ds(off, k)], srcv[s * grp + j], isem[s])
                pltpu.async_copy(dst_hbm.at[pl.ds(off, k)], dstv[s * grp + j], isem[s])

        def drain_idx(s):
            for j in range(grp):
                pltpu.make_async_copy(src_hbm.at[pl.ds(0, k)], srcv[s * grp + j], isem[s]).wait()
                pltpu.make_async_copy(dst_hbm.at[pl.ds(0, k)], dstv[s * grp + j], isem[s]).wait()

        def fire_gather(s):
            for j in range(grp):
                pltpu.async_copy(rows_hbm.at[srcv[s * grp + j]], bufs[s * grp + j], gsem[s])

        def drain_gather(s):
            for j in range(grp):
                pltpu.make_async_copy(rows_hbm.at[srcv[s * grp + j]], bufs[s * grp + j], gsem[s]).wait()

        def scatters(s):
            for j in range(grp):
                pltpu.sync_copy(bufs[s * grp + j], acc_sh.at[dstv[s * grp + j]], add=True)

        pltpu.sync_copy(zeros_hbm.at[pl.ds(rbase, rpt)], acc_sh.at[pl.ds(rbase, rpt)])
        plsc.subcore_barrier()

        fire_idx(0, 0)
        fire_idx(1, 1)
        drain_idx(0)
        fire_gather(0)

        def group(n, p, has_next, has_next2):
            if has_next:
                drain_idx(1 - p)
                fire_gather(1 - p)
            drain_gather(p)
            scatters(p)
            if has_next2:
                fire_idx(n + 2, p)

        def body(t, carry):
            group(2 * t, 0, True, True)
            group(2 * t + 1, 1, True, True)
            return carry

        nt = 3 if ng % 2 == 1 else 4
        lax.fori_loop(0, (ng - nt) // 2, body, 0)
        n0 = ng - nt
        for i in range(nt):
            n = n0 + i
            group(n, n % 2, i < nt - 1, i < nt - 2)

        plsc.subcore_barrier()
        pltpu.sync_copy(acc_sh.at[pl.ds(rbase, rpt)],
                        out_hbm.at[cid, pl.ds(rbase, rpt)])

    return scat_kernel


def _tc1_body(x_ref, w_ref, d0_ref, d1_ref, xws_ref, dinv_ref):
    deg = d0_ref[...] + d1_ref[...] + 1.0
    dv = lax.rsqrt(deg)
    xw = jnp.dot(x_ref[...], w_ref[...], preferred_element_type=jnp.float32)
    xws_ref[...] = xw * dv[:, 0:1]
    dinv_ref[...] = dv


def _tc1(n, d_in, d_hid, bt):
    return pl.pallas_call(
        _tc1_body,
        grid=(n // bt,),
        in_specs=[
            pl.BlockSpec((bt, d_in), lambda i: (i, 0)),
            pl.BlockSpec((d_in, d_hid), lambda i: (0, 0)),
            pl.BlockSpec((bt, DEGW), lambda i: (i, 0)),
            pl.BlockSpec((bt, DEGW), lambda i: (i, 0)),
        ],
        out_specs=[
            pl.BlockSpec((bt, d_hid), lambda i: (i, 0)),
            pl.BlockSpec((bt, DEGW), lambda i: (i, 0)),
        ],
        out_shape=[
            jax.ShapeDtypeStruct((n, d_hid), jnp.float32),
            jax.ShapeDtypeStruct((n, DEGW), jnp.float32),
        ],
    )


def _tc2_body(p0_ref, p1_ref, xws_ref, dinv_ref, b1_ref, wcat_ref, out_ref):
    dv = dinv_ref[...][:, 0:1]
    s = (p0_ref[...] + p1_ref[...] + xws_ref[...]) * dv + b1_ref[...]
    h = jnp.maximum(s, 0.0)
    out_ref[...] = jnp.dot(h, wcat_ref[...], preferred_element_type=jnp.float32) * dv


def _tc2(n, d_hid, d_cat, bt):
    return pl.pallas_call(
        _tc2_body,
        grid=(n // bt,),
        in_specs=[
            pl.BlockSpec((bt, d_hid), lambda i: (i, 0)),
            pl.BlockSpec((bt, d_hid), lambda i: (i, 0)),
            pl.BlockSpec((bt, d_hid), lambda i: (i, 0)),
            pl.BlockSpec((bt, DEGW), lambda i: (i, 0)),
            pl.BlockSpec((1, d_hid), lambda i: (0, 0)),
            pl.BlockSpec((d_hid, d_cat), lambda i: (0, 0)),
        ],
        out_specs=pl.BlockSpec((bt, d_cat), lambda i: (i, 0)),
        out_shape=jax.ShapeDtypeStruct((n, d_cat), jnp.float32),
    )


def _tc3_body(q0_ref, q1_ref, hcs_ref, dinv_ref, bm_ref, bl_ref, mu_ref, ls_ref):
    d_lat = mu_ref.shape[1]
    dv = dinv_ref[...][:, 0:1]
    o = (q0_ref[...] + q1_ref[...] + hcs_ref[...]) * dv
    mu_ref[...] = o[:, :d_lat] + bm_ref[...]
    ls_ref[...] = o[:, d_lat:] + bl_ref[...]


def _tc3(n, d_cat, d_lat, bt):
    return pl.pallas_call(
        _tc3_body,
        grid=(n // bt,),
        in_specs=[
            pl.BlockSpec((bt, d_cat), lambda i: (i, 0)),
            pl.BlockSpec((bt, d_cat), lambda i: (i, 0)),
            pl.BlockSpec((bt, d_cat), lambda i: (i, 0)),
            pl.BlockSpec((bt, DEGW), lambda i: (i, 0)),
            pl.BlockSpec((1, d_lat), lambda i: (0, 0)),
            pl.BlockSpec((1, d_lat), lambda i: (0, 0)),
        ],
        out_specs=[
            pl.BlockSpec((bt, d_lat), lambda i: (i, 0)),
            pl.BlockSpec((bt, d_lat), lambda i: (i, 0)),
        ],
        out_shape=[
            jax.ShapeDtypeStruct((n, d_lat), jnp.float32),
            jax.ShapeDtypeStruct((n, d_lat), jnp.float32),
        ],
    )


def kernel(x, edge_index, W1, b1, W_mu, b_mu, W_logstd, b_logstd):
    n, d_in = x.shape
    d_hid = W1.shape[1]
    d_lat = W_mu.shape[1]
    d_cat = 2 * d_lat
    e = edge_index.shape[1]
    bt = 1000
    gran = 8 * NS
    npad = ((n + gran - 1) // gran) * gran

    kc = 80
    epw_pad = ((e // NW + 2 * kc - 1) // (2 * kc)) * (2 * kc)
    e_pad = epw_pad * NW
    pad = e_pad - e
    src = jnp.concatenate([edge_index[0].astype(jnp.int32),
                           jnp.zeros((pad,), jnp.int32)])
    dst = jnp.concatenate([edge_index[1].astype(jnp.int32),
                           n + (jnp.arange(pad, dtype=jnp.int32) % (npad - n))])

    ones_k = jnp.ones((kc, DEGW), jnp.float32)
    zeros_deg = jnp.zeros((npad, DEGW), jnp.float32)
    degp = _sc_degree(npad, e_pad, kc, 6)(dst, ones_k, zeros_deg)

    xws, dinv = _tc1(n, d_in, d_hid, bt)(x, W1, degp[0, :n], degp[1, :n])

    zeros_h = jnp.zeros((npad, d_hid), jnp.float32)
    p = _sc_scatter(npad, e_pad, d_hid, kc, 2)(src, dst, xws, zeros_h)

    wcat = jnp.concatenate([W_mu, W_logstd], axis=1)
    hcs = _tc2(n, d_hid, d_cat, bt)(p[0, :n], p[1, :n], xws, dinv,
                                    b1.reshape(1, d_hid), wcat)

    zeros_c = jnp.zeros((npad, d_cat), jnp.float32)
    q = _sc_scatter(npad, e_pad, d_cat, kc, 6)(src, dst, hcs, zeros_c)

    mu, logstd = _tc3(n, d_cat, d_lat, bt)(q[0, :n], q[1, :n], hcs, dinv,
                                           b_mu.reshape(1, d_lat),
                                           b_logstd.reshape(1, d_lat))
    return (mu, logstd)


# speedup vs baseline: 2.0219x; 1.2762x over previous
import functools

import jax
import jax.numpy as jnp
from jax import lax
from jax.experimental import pallas as pl
from jax.experimental.pallas import tpu as pltpu
from jax.experimental.pallas import tpu_sc as plsc

NC = 2
NS = 16
NW = NC * NS
DEGW = 16


def _sc_mesh():
    return plsc.VectorSubcoreMesh(core_axis_name="c", subcore_axis_name="s",
                                  num_cores=NC, num_subcores=NS)


def _sc_degree(npad, e, k, grp):
    epw = e // NW
    ch = epw // k
    rpt = npad // NS

    ng = ch // grp
    assert ng * grp == ch and ng >= 2

    scratch = (
        [pltpu.VMEM((k,), jnp.int32) for _ in range(2 * grp)]
        + [pltpu.VMEM((k, DEGW), jnp.float32),
           pltpu.VMEM_SHARED((npad, DEGW), jnp.float32)]
        + [pltpu.SemaphoreType.DMA] * 2
    )

    @functools.partial(
        pl.kernel, mesh=_sc_mesh(),
        out_type=jax.ShapeDtypeStruct((NC, npad, DEGW), jnp.float32),
        compiler_params=pltpu.CompilerParams(use_tc_tiling_on_sc=False),
        scratch_types=scratch,
    )
    def deg_kernel(dst_hbm, ones_hbm, zeros_hbm, out_hbm, *refs):
        dstv = refs[0:2 * grp]
        ones_v = refs[2 * grp]
        acc_sh = refs[2 * grp + 1]
        isem = refs[2 * grp + 2:2 * grp + 4]
        cid = lax.axis_index("c")
        sid = lax.axis_index("s")
        wid = cid * NS + sid
        rbase = sid * rpt
        ebase = wid * epw

        def fire_idx(n, s):
            for j in range(grp):
                off = ebase + (n * grp + j) * k
                pltpu.async_copy(dst_hbm.at[pl.ds(off, k)], dstv[s * grp + j], isem[s])

        def drain_idx(s):
            for j in range(grp):
                pltpu.make_async_copy(dst_hbm.at[pl.ds(0, k)], dstv[s * grp + j], isem[s]).wait()

        def scatters(s):
            for j in range(grp):
                pltpu.sync_copy(ones_v, acc_sh.at[dstv[s * grp + j]], add=True)

        pltpu.sync_copy(ones_hbm, ones_v)
        pltpu.sync_copy(zeros_hbm.at[pl.ds(rbase, rpt)], acc_sh.at[pl.ds(rbase, rpt)])
        plsc.subcore_barrier()

        def group(n, p, has_next):
            if has_next:
                fire_idx(n + 1, 1 - p)
            drain_idx(p)
            scatters(p)

        fire_idx(0, 0)

        def body(t, carry):
            group(2 * t, 0, True)
            group(2 * t + 1, 1, True)
            return carry

        nt = 1 if ng % 2 == 1 else 2
        lax.fori_loop(0, (ng - nt) // 2, body, 0)
        for i in range(nt):
            n = ng - nt + i
            group(n, n % 2, i < nt - 1)

        plsc.subcore_barrier()
        pltpu.sync_copy(acc_sh.at[pl.ds(rbase, rpt)],
                        out_hbm.at[cid, pl.ds(rbase, rpt)])

    return deg_kernel


def _sc_scatter(npad, e, d, k, grp):
    epw = e // NW
    ch = epw // k
    ng = ch // grp
    assert ng * grp == ch and ng >= 4
    rpt = npad // NS

    scratch = (
        [pltpu.VMEM((k,), jnp.int32) for _ in range(2 * grp)]
        + [pltpu.VMEM((k,), jnp.int32) for _ in range(2 * grp)]
        + [pltpu.VMEM((k, d), jnp.float32) for _ in range(2 * grp)]
        + [pltpu.VMEM_SHARED((npad, d), jnp.float32)]
        + [pltpu.SemaphoreType.DMA] * 4
    )

    @functools.partial(
        pl.kernel, mesh=_sc_mesh(),
        out_type=jax.ShapeDtypeStruct((NC, npad, d), jnp.float32),
        compiler_params=pltpu.CompilerParams(use_tc_tiling_on_sc=(d % 128 == 0)),
        scratch_types=scratch,
    )
    def scat_kernel(src_hbm, dst_hbm, rows_hbm, zeros_hbm, out_hbm, *refs):
        srcv = refs[0:2 * grp]
        dstv = refs[2 * grp:4 * grp]
        bufs = refs[4 * grp:6 * grp]
        acc_sh = refs[6 * grp]
        isem = refs[6 * grp + 1:6 * grp + 3]
        gsem = refs[6 * grp + 3:6 * grp + 5]
        cid = lax.axis_index("c")
        sid = lax.axis_index("s")
        wid = cid * NS + sid
        rbase = sid * rpt
        ebase = wid * epw

        def fire_idx(n, s):
            for j in range(grp):
                off = ebase + (n * grp + j) * k
                pltpu.async_copy(src_hbm.at[pl.ds(off, k)], srcv[s * grp + j], isem[s])
                pltpu.async_copy(dst_hbm.at[pl.ds(off, k)], dstv[s * grp + j], isem[s])

        def drain_idx(s):
            for j in range(grp):
                pltpu.make_async_copy(src_hbm.at[pl.ds(0, k)], srcv[s * grp + j], isem[s]).wait()
                pltpu.make_async_copy(dst_hbm.at[pl.ds(0, k)], dstv[s * grp + j], isem[s]).wait()

        def fire_gather(s):
            for j in range(grp):
                pltpu.async_copy(rows_hbm.at[srcv[s * grp + j]], bufs[s * grp + j], gsem[s])

        def drain_gather(s):
            for j in range(grp):
                pltpu.make_async_copy(rows_hbm.at[srcv[s * grp + j]], bufs[s * grp + j], gsem[s]).wait()

        def scatters(s):
            for j in range(grp):
                pltpu.sync_copy(bufs[s * grp + j], acc_sh.at[dstv[s * grp + j]], add=True)

        pltpu.sync_copy(zeros_hbm.at[pl.ds(rbase, rpt)], acc_sh.at[pl.ds(rbase, rpt)])
        plsc.subcore_barrier()

        fire_idx(0, 0)
        fire_idx(1, 1)
        drain_idx(0)
        fire_gather(0)

        def group(n, p, has_next, has_next2):
            if has_next:
                drain_idx(1 - p)
                fire_gather(1 - p)
            drain_gather(p)
            scatters(p)
            if has_next2:
                fire_idx(n + 2, p)

        def body(t, carry):
            group(2 * t, 0, True, True)
            group(2 * t + 1, 1, True, True)
            return carry

        nt = 3 if ng % 2 == 1 else 4
        lax.fori_loop(0, (ng - nt) // 2, body, 0)
        n0 = ng - nt
        for i in range(nt):
            n = n0 + i
            group(n, n % 2, i < nt - 1, i < nt - 2)

        plsc.subcore_barrier()
        pltpu.sync_copy(acc_sh.at[pl.ds(rbase, rpt)],
                        out_hbm.at[cid, pl.ds(rbase, rpt)])

    return scat_kernel


def _tc1_body(x_ref, w_ref, d0_ref, d1_ref, xws_ref, dinv_ref):
    deg = d0_ref[...] + d1_ref[...] + 1.0
    dv = lax.rsqrt(deg)
    xw = jnp.dot(x_ref[...], w_ref[...], preferred_element_type=jnp.float32)
    xws_ref[...] = xw * dv[:, 0:1]
    dinv_ref[...] = dv


def _tc1(n, d_in, d_hid, bt):
    return pl.pallas_call(
        _tc1_body,
        grid=(n // bt,),
        in_specs=[
            pl.BlockSpec((bt, d_in), lambda i: (i, 0)),
            pl.BlockSpec((d_in, d_hid), lambda i: (0, 0)),
            pl.BlockSpec((bt, DEGW), lambda i: (i, 0)),
            pl.BlockSpec((bt, DEGW), lambda i: (i, 0)),
        ],
        out_specs=[
            pl.BlockSpec((bt, d_hid), lambda i: (i, 0)),
            pl.BlockSpec((bt, DEGW), lambda i: (i, 0)),
        ],
        out_shape=[
            jax.ShapeDtypeStruct((n, d_hid), jnp.float32),
            jax.ShapeDtypeStruct((n, DEGW), jnp.float32),
        ],
    )


def _tc2_body(p0_ref, p1_ref, xws_ref, dinv_ref, b1_ref, wcat_ref, out_ref):
    dv = dinv_ref[...][:, 0:1]
    s = (p0_ref[...] + p1_ref[...] + xws_ref[...]) * dv + b1_ref[...]
    h = jnp.maximum(s, 0.0)
    out_ref[...] = jnp.dot(h, wcat_ref[...], preferred_element_type=jnp.float32) * dv


def _tc2(n, d_hid, d_cat, bt):
    return pl.pallas_call(
        _tc2_body,
        grid=(n // bt,),
        in_specs=[
            pl.BlockSpec((bt, d_hid), lambda i: (i, 0)),
            pl.BlockSpec((bt, d_hid), lambda i: (i, 0)),
            pl.BlockSpec((bt, d_hid), lambda i: (i, 0)),
            pl.BlockSpec((bt, DEGW), lambda i: (i, 0)),
            pl.BlockSpec((1, d_hid), lambda i: (0, 0)),
            pl.BlockSpec((d_hid, d_cat), lambda i: (0, 0)),
        ],
        out_specs=pl.BlockSpec((bt, d_cat), lambda i: (i, 0)),
        out_shape=jax.ShapeDtypeStruct((n, d_cat), jnp.float32),
    )


def _tc3_body(q0_ref, q1_ref, hcs_ref, dinv_ref, bm_ref, bl_ref, mu_ref, ls_ref):
    d_lat = mu_ref.shape[1]
    dv = dinv_ref[...][:, 0:1]
    o = (q0_ref[...] + q1_ref[...] + hcs_ref[...]) * dv
    mu_ref[...] = o[:, :d_lat] + bm_ref[...]
    ls_ref[...] = o[:, d_lat:] + bl_ref[...]


def _tc3(n, d_cat, d_lat, bt):
    return pl.pallas_call(
        _tc3_body,
        grid=(n // bt,),
        in_specs=[
            pl.BlockSpec((bt, d_cat), lambda i: (i, 0)),
            pl.BlockSpec((bt, d_cat), lambda i: (i, 0)),
            pl.BlockSpec((bt, d_cat), lambda i: (i, 0)),
            pl.BlockSpec((bt, DEGW), lambda i: (i, 0)),
            pl.BlockSpec((1, d_lat), lambda i: (0, 0)),
            pl.BlockSpec((1, d_lat), lambda i: (0, 0)),
        ],
        out_specs=[
            pl.BlockSpec((bt, d_lat), lambda i: (i, 0)),
            pl.BlockSpec((bt, d_lat), lambda i: (i, 0)),
        ],
        out_shape=[
            jax.ShapeDtypeStruct((n, d_lat), jnp.float32),
            jax.ShapeDtypeStruct((n, d_lat), jnp.float32),
        ],
    )


def kernel(x, edge_index, W1, b1, W_mu, b_mu, W_logstd, b_logstd):
    n, d_in = x.shape
    d_hid = W1.shape[1]
    d_lat = W_mu.shape[1]
    d_cat = 2 * d_lat
    e = edge_index.shape[1]
    bt = 1000
    gran = 8 * NS
    npad = ((n + gran - 1) // gran) * gran

    kc = 40
    epw_pad = ((e // NW + 2 * kc - 1) // (2 * kc)) * (2 * kc)
    e_pad = epw_pad * NW
    pad = e_pad - e
    src = jnp.concatenate([edge_index[0].astype(jnp.int32),
                           jnp.zeros((pad,), jnp.int32)])
    dst = jnp.concatenate([edge_index[1].astype(jnp.int32),
                           n + (jnp.arange(pad, dtype=jnp.int32) % (npad - n))])

    ones_k = jnp.ones((80, DEGW), jnp.float32)
    zeros_deg = jnp.zeros((npad, DEGW), jnp.float32)
    degp = _sc_degree(npad, e_pad, 80, 5)(dst, ones_k, zeros_deg)

    xws, dinv = _tc1(n, d_in, d_hid, bt)(x, W1, degp[0, :n], degp[1, :n])

    zeros_h = jnp.zeros((npad, d_hid), jnp.float32)
    p = _sc_scatter(npad, e_pad, d_hid, kc, 2)(src, dst, xws, zeros_h)

    wcat = jnp.concatenate([W_mu, W_logstd], axis=1)
    hcs = _tc2(n, d_hid, d_cat, bt)(p[0, :n], p[1, :n], xws, dinv,
                                    b1.reshape(1, d_hid), wcat)

    zeros_c = jnp.zeros((npad, d_cat), jnp.float32)
    q = _sc_scatter(npad, e_pad, d_cat, 80, 5)(src, dst, hcs, zeros_c)

    mu, logstd = _tc3(n, d_cat, d_lat, bt)(q[0, :n], q[1, :n], hcs, dinv,
                                           b_mu.reshape(1, d_lat),
                                           b_logstd.reshape(1, d_lat))
    return (mu, logstd)
